# baseline (speedup 1.0000x reference)
import functools

import jax
import jax.numpy as jnp
from jax import lax
from jax.experimental import pallas as pl
from jax.experimental.pallas import tpu as pltpu
from jax.experimental.pallas import tpu_sc as plsc

N_LEMMAS = 100000
NPH = 128
NCD = 128
B = 1024
NEG = -1e9

LB = 2048
GRID = (N_LEMMAS + LB - 1) // LB

NW = 32
BPW = B // NW
SCAT_CHUNK = 8


def _prep_body(idx_r_ref, idx_c_ref, cv_ref, pc_ref, cv2_ref, pc2_ref, pos_ref):
    ir = idx_r_ref[...]
    ic = idx_c_ref[...]
    same = ir == ic
    iota_c = lax.broadcasted_iota(jnp.int32, (B, B), 1)
    last = jnp.max(jnp.where(same, iota_c, -1), axis=1, keepdims=True)
    iota_r = lax.broadcasted_iota(jnp.int32, (B, 1), 0)
    is_last = last == iota_r
    onehot = (iota_c == last).astype(jnp.bfloat16)
    cv_sel = jnp.dot(onehot, cv_ref[...].astype(jnp.bfloat16),
                     preferred_element_type=jnp.float32)
    pc_sel = jnp.dot(onehot, pc_ref[...].astype(jnp.bfloat16),
                     preferred_element_type=jnp.float32)
    cv2_ref[...] = jnp.where(is_last, cv_ref[...], cv_sel)
    pc2_ref[...] = jnp.where(is_last, pc_ref[...], pc_sel)
    pos_ref[...] = ir + lax.broadcasted_iota(jnp.int32, (B, NPH), 1) * N_LEMMAS


_prep = pl.pallas_call(
    _prep_body,
    out_shape=(
        jax.ShapeDtypeStruct((B, NCD), jnp.float32),
        jax.ShapeDtypeStruct((B, NPH), jnp.float32),
        jax.ShapeDtypeStruct((B, NPH), jnp.int32),
    ),
)


def _sims_body(pc_ref, w_ref, st_ref, out_ref, inp_ref, acc_ref):
    i = pl.program_id(0)

    @pl.when(i == 0)
    def _():
        pc = pc_ref[...]
        nrm = jnp.sqrt(jnp.sum(pc * pc, axis=1, keepdims=True))
        inp_ref[...] = (pc / (nrm + 1e-8)).astype(jnp.bfloat16)

    w = w_ref[...]
    nsq = jnp.sum(w * w, axis=0, keepdims=True)
    inv = 1.0 / (jnp.sqrt(nsq) + 1e-8)
    lane = lax.broadcasted_iota(jnp.int32, (1, LB), 1) + i * LB
    valid = lane < N_LEMMAS
    wn = jnp.where(valid, w * inv, 0.0).astype(jnp.bfloat16)
    dot = jnp.dot(inp_ref[...], wn, preferred_element_type=jnp.float32)
    bias = jnp.where(valid & (st_ref[...] > 0), 0.0, NEG)
    part = dot + bias

    @pl.when(i == 0)
    def _():
        acc_ref[...] = part

    @pl.when(i > 0)
    def _():
        acc_ref[...] = jnp.maximum(acc_ref[...], part)

    @pl.when(i == GRID - 1)
    def _():
        out_ref[...] = jnp.max(acc_ref[...], axis=1, keepdims=True)


_sims = pl.pallas_call(
    _sims_body,
    grid=(GRID,),
    in_specs=[
        pl.BlockSpec((B, NPH), lambda i: (0, 0)),
        pl.BlockSpec((NPH, LB), lambda i: (0, i)),
        pl.BlockSpec((1, LB), lambda i: (0, i)),
    ],
    out_specs=pl.BlockSpec((B, 1), lambda i: (0, 0)),
    out_shape=jax.ShapeDtypeStruct((B, 1), jnp.float32),
    scratch_shapes=[
        pltpu.VMEM((B, NPH), jnp.bfloat16),
        pltpu.VMEM((B, LB), jnp.float32),
    ],
    compiler_params=pltpu.CompilerParams(
        dimension_semantics=("arbitrary",),
    ),
)


def _sc_scatter_body(idx_hbm, cv2_hbm, pos_hbm, pc2_hbm, wcl_ref, wlp_ref,
                     idx_v, row_v, pos_v, val_v, sem_row, sem_col):
    wid = lax.axis_index("s") * 2 + lax.axis_index("c")
    base = wid * BPW
    pltpu.sync_copy(idx_hbm.at[pl.ds(base, BPW)], idx_v)
    pltpu.sync_copy(cv2_hbm.at[pl.ds(base, BPW)], row_v)
    pltpu.sync_copy(pos_hbm.at[pl.ds(base, BPW)], pos_v)
    pltpu.sync_copy(pc2_hbm.at[pl.ds(base, BPW)], val_v)
    row_copy = pltpu.async_copy(row_v, wcl_ref.at[idx_v], sem_row)
    for j0 in range(0, BPW, SCAT_CHUNK):
        copies = [
            pltpu.async_copy(val_v.at[j], wlp_ref.at[pos_v.at[j]], sem_col)
            for j in range(j0, j0 + SCAT_CHUNK)
        ]
        for c in copies:
            c.wait()
    row_copy.wait()


@functools.cache
def _sc_scatter():
    return pl.kernel(
        _sc_scatter_body,
        mesh=plsc.VectorSubcoreMesh(core_axis_name="c", subcore_axis_name="s",
                                    num_cores=2, num_subcores=16),
        scratch_types=[
            pltpu.VMEM((BPW,), jnp.int32),
            pltpu.VMEM((BPW, NCD), jnp.float32),
            pltpu.VMEM((BPW, NPH), jnp.int32),
            pltpu.VMEM((BPW, NPH), jnp.float32),
            pltpu.SemaphoreType.DMA,
            pltpu.SemaphoreType.DMA,
        ],
    )


def kernel(concept_vector, phonological_code, idx, W_C_to_L, W_L_to_P, status):
    cv2, pc2, pos = _prep(
        idx.reshape(B, 1), idx.reshape(1, B), concept_vector, phonological_code)
    maxsim = _sims(phonological_code, W_L_to_P, status.reshape(1, N_LEMMAS))
    wcl_ref = jax.new_ref(W_C_to_L)
    wlp_ref = jax.new_ref(W_L_to_P.reshape(-1))
    _sc_scatter()(idx, cv2, pos, pc2, wcl_ref, wlp_ref)
    return (wcl_ref[...],
            wlp_ref[...].reshape(NPH, N_LEMMAS),
            maxsim.reshape(B))


# speedup vs baseline: 1.2473x; 1.2473x over previous
import functools

import jax
import jax.numpy as jnp
from jax import lax
from jax.experimental import pallas as pl
from jax.experimental.pallas import tpu as pltpu
from jax.experimental.pallas import tpu_sc as plsc

N_LEMMAS = 100000
NPH = 128
NCD = 128
B = 1024
NEG = -1e9

LB = 2048
GRID = (N_LEMMAS + LB - 1) // LB
MPAD = 136

NW = 32
BPW = B // NW


def _prep_body(idx_r_ref, idx_c_ref, cv_ref, cv2_ref, islast_ref):
    ir = idx_r_ref[...]
    ic = idx_c_ref[...]
    same = ir == ic
    iota_c = lax.broadcasted_iota(jnp.int32, (B, B), 1)
    last = jnp.max(jnp.where(same, iota_c, -1), axis=1, keepdims=True)
    iota_r = lax.broadcasted_iota(jnp.int32, (B, 1), 0)
    is_last = last == iota_r
    onehot = (iota_c == last).astype(jnp.bfloat16)
    cv_sel = jnp.dot(onehot, cv_ref[...].astype(jnp.bfloat16),
                     preferred_element_type=jnp.float32)
    cv2_ref[...] = jnp.where(is_last, cv_ref[...], cv_sel)
    islast_ref[...] = is_last.astype(jnp.float32)


_prep = pl.pallas_call(
    _prep_body,
    out_shape=(
        jax.ShapeDtypeStruct((B, NCD), jnp.float32),
        jax.ShapeDtypeStruct((B, 1), jnp.float32),
    ),
)


def _sims_body(pc_ref, pcT_ref, idx_r_ref, islast_ref, w_ref, st_ref,
               wout_ref, ms_ref, inp_ref, pcTe_ref, acc_ref):
    i = pl.program_id(0)

    @pl.when(i == 0)
    def _():
        pc = pc_ref[...]
        nrm = jnp.sqrt(jnp.sum(pc * pc, axis=1, keepdims=True))
        inp_ref[...] = (pc / (nrm + 1e-8)).astype(jnp.bfloat16)
        pcTe_ref[...] = jnp.concatenate(
            [pcT_ref[...].astype(jnp.bfloat16),
             jnp.ones((1, B), jnp.bfloat16),
             jnp.zeros((MPAD - NPH - 1, B), jnp.bfloat16)], axis=0)

    lane = lax.broadcasted_iota(jnp.int32, (1, LB), 1) + i * LB
    valid = lane < N_LEMMAS
    w = w_ref[...]
    nsq = jnp.sum(w * w, axis=0, keepdims=True)
    inv = 1.0 / (jnp.sqrt(nsq) + 1e-8)
    wn = jnp.where(valid, w * inv, 0.0).astype(jnp.bfloat16)
    dot = jnp.dot(inp_ref[...], wn, preferred_element_type=jnp.float32)
    bias = jnp.where(valid & (st_ref[...] > 0), 0.0, NEG)
    part = dot + bias

    @pl.when(i == 0)
    def _():
        acc_ref[...] = part

    @pl.when(i > 0)
    def _():
        acc_ref[...] = jnp.maximum(acc_ref[...], part)

    oh = ((idx_r_ref[...] == lane) & (islast_ref[...] > 0.5)).astype(jnp.bfloat16)
    scat_ext = jnp.dot(pcTe_ref[...], oh, preferred_element_type=jnp.float32)
    scat = scat_ext[0:NPH, :]
    cnt = scat_ext[NPH:NPH + 1, :]
    wout_ref[...] = jnp.where(cnt > 0.5, scat, w)

    @pl.when(i == GRID - 1)
    def _():
        ms_ref[...] = jnp.max(acc_ref[...], axis=1, keepdims=True)


_sims = pl.pallas_call(
    _sims_body,
    grid=(GRID,),
    in_specs=[
        pl.BlockSpec((B, NPH), lambda i: (0, 0)),
        pl.BlockSpec((NPH, B), lambda i: (0, 0)),
        pl.BlockSpec((B, 1), lambda i: (0, 0)),
        pl.BlockSpec((B, 1), lambda i: (0, 0)),
        pl.BlockSpec((NPH, LB), lambda i: (0, i)),
        pl.BlockSpec((1, LB), lambda i: (0, i)),
    ],
    out_specs=(
        pl.BlockSpec((NPH, LB), lambda i: (0, i)),
        pl.BlockSpec((B, 1), lambda i: (0, 0)),
    ),
    out_shape=(
        jax.ShapeDtypeStruct((NPH, N_LEMMAS), jnp.float32),
        jax.ShapeDtypeStruct((B, 1), jnp.float32),
    ),
    scratch_shapes=[
        pltpu.VMEM((B, NPH), jnp.bfloat16),
        pltpu.VMEM((MPAD, B), jnp.bfloat16),
        pltpu.VMEM((B, LB), jnp.float32),
    ],
    compiler_params=pltpu.CompilerParams(
        dimension_semantics=("arbitrary",),
    ),
)


def _sc_scatter_body(idx_hbm, cv2_hbm, wcl_ref, idx_v, row_v, sem_row):
    wid = lax.axis_index("s") * 2 + lax.axis_index("c")
    base = wid * BPW
    pltpu.sync_copy(idx_hbm.at[pl.ds(base, BPW)], idx_v)
    pltpu.sync_copy(cv2_hbm.at[pl.ds(base, BPW)], row_v)
    pltpu.async_copy(row_v, wcl_ref.at[idx_v], sem_row).wait()


@functools.cache
def _sc_scatter():
    return pl.kernel(
        _sc_scatter_body,
        mesh=plsc.VectorSubcoreMesh(core_axis_name="c", subcore_axis_name="s",
                                    num_cores=2, num_subcores=16),
        scratch_types=[
            pltpu.VMEM((BPW,), jnp.int32),
            pltpu.VMEM((BPW, NCD), jnp.float32),
            pltpu.SemaphoreType.DMA,
        ],
    )


def kernel(concept_vector, phonological_code, idx, W_C_to_L, W_L_to_P, status):
    cv2, islast = _prep(idx.reshape(B, 1), idx.reshape(1, B), concept_vector)
    wlp_out, maxsim = _sims(phonological_code, phonological_code.T,
                            idx.reshape(B, 1), islast, W_L_to_P,
                            status.reshape(1, N_LEMMAS))
    wcl_ref = jax.new_ref(W_C_to_L)
    _sc_scatter()(idx, cv2, wcl_ref)
    return wcl_ref[...], wlp_out, maxsim.reshape(B)


# speedup vs baseline: 2.7393x; 2.1961x over previous
import functools

import jax
import jax.numpy as jnp
from jax import lax
from jax.experimental import pallas as pl
from jax.experimental.pallas import tpu as pltpu
from jax.experimental.pallas import tpu_sc as plsc

N_LEMMAS = 100000
NPH = 128
NCD = 128
B = 1024
NEG = -1e9

LB = 2048
GRID = (N_LEMMAS + LB - 1) // LB

NW = 32
BPW = B // NW


def _prep_body(idx_r_ref, idx_c_ref, cv_ref, pc_ref, cv2_ref, pc2_ref):
    ir = idx_r_ref[...]
    ic = idx_c_ref[...]
    same = ir == ic
    iota_c = lax.broadcasted_iota(jnp.int32, (B, B), 1)
    last = jnp.max(jnp.where(same, iota_c, -1), axis=1, keepdims=True)
    iota_r = lax.broadcasted_iota(jnp.int32, (B, 1), 0)
    is_last = last == iota_r
    onehot = (iota_c == last).astype(jnp.bfloat16)
    cv_sel = jnp.dot(onehot, cv_ref[...].astype(jnp.bfloat16),
                     preferred_element_type=jnp.float32)
    pc_sel = jnp.dot(onehot, pc_ref[...].astype(jnp.bfloat16),
                     preferred_element_type=jnp.float32)
    cv2_ref[...] = jnp.where(is_last, cv_ref[...], cv_sel)
    pc2_ref[...] = jnp.where(is_last, pc_ref[...], pc_sel)


_prep = pl.pallas_call(
    _prep_body,
    out_shape=(
        jax.ShapeDtypeStruct((B, NCD), jnp.float32),
        jax.ShapeDtypeStruct((B, NPH), jnp.float32),
    ),
)


def _sims_body(pc_ref, wt_ref, st_ref, ms_ref, inp_ref, ones_ref, acc_ref):
    i = pl.program_id(0)

    @pl.when(i == 0)
    def _():
        pc = pc_ref[...]
        nrm = jnp.sqrt(jnp.sum(pc * pc, axis=1, keepdims=True))
        inp_ref[...] = (pc / (nrm + 1e-8)).astype(jnp.bfloat16)
        ones_ref[...] = jnp.ones((NPH, 8), jnp.bfloat16)
        acc_ref[...] = jnp.full((B, LB), NEG, jnp.float32)

    w = wt_ref[...]
    row = lax.broadcasted_iota(jnp.int32, (LB, 1), 0) + i * LB
    rvalid = row < N_LEMMAS
    wsq = (w * w).astype(jnp.bfloat16)
    nsq = jnp.dot(wsq, ones_ref[...],
                  preferred_element_type=jnp.float32)[:, 0:1]
    inv = 1.0 / (jnp.sqrt(nsq) + 1e-8)
    wn = jnp.where(rvalid, w * inv, 0.0).astype(jnp.bfloat16)
    dot = lax.dot_general(inp_ref[...], wn, (((1,), (1,)), ((), ())),
                          preferred_element_type=jnp.float32)
    lane = lax.broadcasted_iota(jnp.int32, (1, LB), 1) + i * LB
    bias = jnp.where((lane < N_LEMMAS) & (st_ref[...] > 0), 0.0, NEG)
    acc_ref[...] = jnp.maximum(acc_ref[...], dot + bias)

    @pl.when(i == GRID - 1)
    def _():
        ms_ref[...] = jnp.max(acc_ref[...], axis=1, keepdims=True)


_sims = pl.pallas_call(
    _sims_body,
    grid=(GRID,),
    in_specs=[
        pl.BlockSpec((B, NPH), lambda i: (0, 0)),
        pl.BlockSpec((LB, NPH), lambda i: (i, 0)),
        pl.BlockSpec((1, LB), lambda i: (0, i)),
    ],
    out_specs=pl.BlockSpec((B, 1), lambda i: (0, 0)),
    out_shape=jax.ShapeDtypeStruct((B, 1), jnp.float32),
    scratch_shapes=[
        pltpu.VMEM((B, NPH), jnp.bfloat16),
        pltpu.VMEM((NPH, 8), jnp.bfloat16),
        pltpu.VMEM((B, LB), jnp.float32),
    ],
    compiler_params=pltpu.CompilerParams(
        dimension_semantics=("arbitrary",),
    ),
)


def _sc_scatter_body(idx_hbm, cv2_hbm, pc2_hbm, wcl_ref, wlpt_ref,
                     idx_v, rowc_v, rowp_v, sem):
    wid = lax.axis_index("s") * 2 + lax.axis_index("c")
    base = wid * BPW
    pltpu.sync_copy(idx_hbm.at[pl.ds(base, BPW)], idx_v)
    pltpu.sync_copy(cv2_hbm.at[pl.ds(base, BPW)], rowc_v)
    pltpu.sync_copy(pc2_hbm.at[pl.ds(base, BPW)], rowp_v)
    c1 = pltpu.async_copy(rowc_v, wcl_ref.at[idx_v], sem)
    c2 = pltpu.async_copy(rowp_v, wlpt_ref.at[idx_v], sem)
    c1.wait()
    c2.wait()


@functools.cache
def _sc_scatter():
    return pl.kernel(
        _sc_scatter_body,
        mesh=plsc.VectorSubcoreMesh(core_axis_name="c", subcore_axis_name="s",
                                    num_cores=2, num_subcores=16),
        scratch_types=[
            pltpu.VMEM((BPW,), jnp.int32),
            pltpu.VMEM((BPW, NCD), jnp.float32),
            pltpu.VMEM((BPW, NPH), jnp.float32),
            pltpu.SemaphoreType.DMA,
        ],
    )


def kernel(concept_vector, phonological_code, idx, W_C_to_L, W_L_to_P, status):
    cv2, pc2 = _prep(idx.reshape(B, 1), idx.reshape(1, B),
                     concept_vector, phonological_code)
    maxsim = _sims(phonological_code, W_L_to_P.T, status.reshape(1, N_LEMMAS))
    wcl_ref = jax.new_ref(W_C_to_L)
    wlpt_ref = jax.new_ref(W_L_to_P.T)
    _sc_scatter()(idx, cv2, pc2, wcl_ref, wlpt_ref)
    return wcl_ref[...], wlpt_ref[...].T, maxsim.reshape(B)


# speedup vs baseline: 3.5813x; 1.3074x over previous
import functools

import jax
import jax.numpy as jnp
from jax import lax
from jax.experimental import pallas as pl
from jax.experimental.pallas import tpu as pltpu
from jax.experimental.pallas import tpu_sc as plsc

N_LEMMAS = 100000
NPH = 128
NCD = 128
B = 1024
NEG = -1e9

LB = 2048
GRID = (N_LEMMAS + LB - 1) // LB

NW = 32
BPW = B // NW


def _prep_body(idx_r_ref, idx_c_ref, cv_ref, pc_ref, cv2_ref, pc2_ref):
    ir = idx_r_ref[...]
    ic = idx_c_ref[...]
    same = ir == ic
    iota_c = lax.broadcasted_iota(jnp.int32, (B, B), 1)
    last = jnp.max(jnp.where(same, iota_c, -1), axis=1, keepdims=True)
    iota_r = lax.broadcasted_iota(jnp.int32, (B, 1), 0)
    is_last = last == iota_r
    onehot = (iota_c == last).astype(jnp.bfloat16)
    cv_sel = jnp.dot(onehot, cv_ref[...].astype(jnp.bfloat16),
                     preferred_element_type=jnp.float32)
    pc_sel = jnp.dot(onehot, pc_ref[...].astype(jnp.bfloat16),
                     preferred_element_type=jnp.float32)
    cv2_ref[...] = jnp.where(is_last, cv_ref[...], cv_sel)
    pc2_ref[...] = jnp.where(is_last, pc_ref[...], pc_sel)


_prep = pl.pallas_call(
    _prep_body,
    out_shape=(
        jax.ShapeDtypeStruct((B, NCD), jnp.float32),
        jax.ShapeDtypeStruct((B, NPH), jnp.float32),
    ),
)


def _sims_body(pc_ref, wt_ref, st_ref, wcl_ref, ms_ref, wt_out_ref, wcl_out_ref,
               inp_ref, ones_ref, acc_ref):
    i = pl.program_id(0)

    @pl.when(i == 0)
    def _():
        pc = pc_ref[...]
        nrm = jnp.sqrt(jnp.sum(pc * pc, axis=1, keepdims=True))
        inp_ref[...] = (pc / (nrm + 1e-8)).astype(jnp.bfloat16)
        ones_ref[...] = jnp.ones((NPH, 8), jnp.bfloat16)
        acc_ref[...] = jnp.full((B, LB), NEG, jnp.float32)

    w = wt_ref[...]
    wt_out_ref[...] = w
    wcl_out_ref[...] = wcl_ref[...]
    row = lax.broadcasted_iota(jnp.int32, (LB, 1), 0) + i * LB
    rvalid = row < N_LEMMAS
    wsq = (w * w).astype(jnp.bfloat16)
    nsq = jnp.dot(wsq, ones_ref[...],
                  preferred_element_type=jnp.float32)[:, 0:1]
    inv = 1.0 / (jnp.sqrt(nsq) + 1e-8)
    wn = jnp.where(rvalid, w * inv, 0.0).astype(jnp.bfloat16)
    dot = lax.dot_general(inp_ref[...], wn, (((1,), (1,)), ((), ())),
                          preferred_element_type=jnp.float32)
    lane = lax.broadcasted_iota(jnp.int32, (1, LB), 1) + i * LB
    bias = jnp.where((lane < N_LEMMAS) & (st_ref[...] > 0), 0.0, NEG)
    acc_ref[...] = jnp.maximum(acc_ref[...], dot + bias)

    @pl.when(i == GRID - 1)
    def _():
        ms_ref[...] = jnp.max(acc_ref[...], axis=1, keepdims=True)


_sims = pl.pallas_call(
    _sims_body,
    grid=(GRID,),
    in_specs=[
        pl.BlockSpec((B, NPH), lambda i: (0, 0)),
        pl.BlockSpec((LB, NPH), lambda i: (i, 0)),
        pl.BlockSpec((1, LB), lambda i: (0, i)),
        pl.BlockSpec((LB, NCD), lambda i: (i, 0)),
    ],
    out_specs=(
        pl.BlockSpec((B, 1), lambda i: (0, 0)),
        pl.BlockSpec((LB, NPH), lambda i: (i, 0)),
        pl.BlockSpec((LB, NCD), lambda i: (i, 0)),
    ),
    out_shape=(
        jax.ShapeDtypeStruct((B, 1), jnp.float32),
        jax.ShapeDtypeStruct((N_LEMMAS, NPH), jnp.float32),
        jax.ShapeDtypeStruct((N_LEMMAS, NCD), jnp.float32),
    ),
    scratch_shapes=[
        pltpu.VMEM((B, NPH), jnp.bfloat16),
        pltpu.VMEM((NPH, 8), jnp.bfloat16),
        pltpu.VMEM((B, LB), jnp.float32),
    ],
    compiler_params=pltpu.CompilerParams(
        dimension_semantics=("arbitrary",),
    ),
)


def _sc_scatter_body(idx_hbm, cv2_hbm, pc2_hbm, wcl_ref, wlpt_ref,
                     idx_v, rowc_v, rowp_v, sem):
    wid = lax.axis_index("s") * 2 + lax.axis_index("c")
    base = wid * BPW
    pltpu.sync_copy(idx_hbm.at[pl.ds(base, BPW)], idx_v)
    pltpu.sync_copy(cv2_hbm.at[pl.ds(base, BPW)], rowc_v)
    pltpu.sync_copy(pc2_hbm.at[pl.ds(base, BPW)], rowp_v)
    c1 = pltpu.async_copy(rowc_v, wcl_ref.at[idx_v], sem)
    c2 = pltpu.async_copy(rowp_v, wlpt_ref.at[idx_v], sem)
    c1.wait()
    c2.wait()


@functools.cache
def _sc_scatter():
    return pl.kernel(
        _sc_scatter_body,
        mesh=plsc.VectorSubcoreMesh(core_axis_name="c", subcore_axis_name="s",
                                    num_cores=2, num_subcores=16),
        scratch_types=[
            pltpu.VMEM((BPW,), jnp.int32),
            pltpu.VMEM((BPW, NCD), jnp.float32),
            pltpu.VMEM((BPW, NPH), jnp.float32),
            pltpu.SemaphoreType.DMA,
        ],
    )


def kernel(concept_vector, phonological_code, idx, W_C_to_L, W_L_to_P, status):
    cv2, pc2 = _prep(idx.reshape(B, 1), idx.reshape(1, B),
                     concept_vector, phonological_code)
    maxsim, wlpt_copy, wcl_copy = _sims(
        phonological_code, W_L_to_P.T, status.reshape(1, N_LEMMAS), W_C_to_L)
    wcl_ref = jax.new_ref(wcl_copy)
    wlpt_ref = jax.new_ref(wlpt_copy)
    _sc_scatter()(idx, cv2, pc2, wcl_ref, wlpt_ref)
    return wcl_ref[...], wlpt_ref[...].T, maxsim.reshape(B)
